# Initial kernel scaffold; baseline (speedup 1.0000x reference)
import jax
import jax.numpy as jnp
from jax.experimental import pallas as pl


def kernel(x, embedding):
    raise NotImplementedError("write your pallas kernel here")


# speedup vs baseline: 3.9091x; 3.9091x over previous
import functools

import jax
import jax.numpy as jnp
from jax import lax
from jax.experimental import pallas as pl
from jax.experimental.pallas import tpu as pltpu
from jax.experimental.pallas import tpu_sc as plsc

_DIM = 32
_CODES = 8192
_TOKENS = 8192
_TOK_BLK = 256
_N_BLK = _TOKENS // _TOK_BLK
_PAD = 128


def _argmin_block(x_ref, e_ref, eslice_ref, idx_ref, et_ref):
    x = x_ref[...]
    e = e_ref[...]
    s = lax.dot_general(x, e, (((1,), (0,)), ((), ())),
                        preferred_element_type=jnp.float32)
    xsq = jnp.sum(x * x, axis=1, keepdims=True)
    esq = jnp.sum(e * e, axis=0, keepdims=True)
    d = (xsq + esq) - 2.0 * s
    dmin = jnp.min(d, axis=1, keepdims=True)
    iota = lax.broadcasted_iota(jnp.int32, d.shape, 1)
    idx = jnp.min(jnp.where(d == dmin, iota, _CODES), axis=1, keepdims=True)
    idx_ref[...] = idx
    et = eslice_ref[...].T
    et_ref[...] = jnp.concatenate(
        [et, jnp.zeros((_TOK_BLK, _PAD - _DIM), jnp.float32)], axis=1)


def _tc_argmin(flat, embedding):
    return pl.pallas_call(
        _argmin_block,
        grid=(_N_BLK,),
        in_specs=[
            pl.BlockSpec((_TOK_BLK, _DIM), lambda i: (i, 0)),
            pl.BlockSpec((_DIM, _CODES), lambda i: (0, 0)),
            pl.BlockSpec((_DIM, _TOK_BLK), lambda i: (0, i)),
        ],
        out_specs=[
            pl.BlockSpec((_TOK_BLK, 1), lambda i: (i, 0)),
            pl.BlockSpec((_TOK_BLK, _PAD), lambda i: (i, 0)),
        ],
        out_shape=[
            jax.ShapeDtypeStruct((_TOKENS, 1), jnp.int32),
            jax.ShapeDtypeStruct((_CODES, _PAD), jnp.float32),
        ],
    )(flat, embedding, embedding)


def _sc_gather(table, idx):
    info = plsc.get_sparse_core_info()
    nw = info.num_cores * info.num_subcores
    b_per_w = _TOKENS // nw
    mesh = plsc.VectorSubcoreMesh(core_axis_name="c", subcore_axis_name="s")

    @functools.partial(
        pl.kernel,
        mesh=mesh,
        out_type=jax.ShapeDtypeStruct((_TOKENS, _PAD), jnp.float32),
        scratch_types=[
            pltpu.VMEM((b_per_w,), jnp.int32),
            pltpu.VMEM((b_per_w, _PAD), jnp.float32),
            pltpu.SemaphoreType.DMA,
        ],
    )
    def gather(table_hbm, idx_hbm, out_hbm, idx_v, rows_v, sem):
        wid = lax.axis_index("s") * info.num_cores + lax.axis_index("c")
        base = wid * b_per_w
        pltpu.sync_copy(idx_hbm.at[pl.ds(base, b_per_w)], idx_v)
        pltpu.async_copy(table_hbm.at[idx_v], rows_v, sem).wait()
        pltpu.sync_copy(rows_v, out_hbm.at[pl.ds(base, b_per_w)])

    return gather(table, idx)


def kernel(x, embedding):
    input_shape = x.shape
    flat = x.reshape(-1, _DIM)
    idx2d, et = _tc_argmin(flat, embedding)
    quantized = _sc_gather(et, idx2d.reshape(-1))
    return quantized[:, :_DIM].reshape(input_shape)
